# baseline (speedup 1.0000x reference)
import functools

import jax
import jax.numpy as jnp
from jax import lax
from jax.experimental import pallas as pl
from jax.experimental.pallas import tpu as pltpu
from jax.experimental.pallas import tpu_sc as plsc

F = 26
D = 32
B = 16384
V = 2600000
NC, NS, L = 2, 16, 16
NW = NC * NS
FPC = F // NC
BPT = B // NS
NGF = BPT // 128

TILE = 128
NT_FULL = V // TILE
W_LAST = V - NT_FULL * TILE
VP = V * D // TILE
GW = 4 * TILE
NGRP = NT_FULL // 4
GPW = -(-NGRP // NW)
VHALF = -(-GPW // 2)


def _transpose_body(tt_hbm, tail_hbm, tlin4_hbm,
                    t4a, t4b, r4a, r4b, tailv, rsa, rsb, wsa, wsb):
    c = lax.axis_index("c")
    s = lax.axis_index("s")
    w = s * NC + c
    iota = lax.iota(jnp.int32, L)

    g0 = w * GPW
    gend = jnp.minimum(g0 + GPW, NGRP)
    n = gend - g0

    def read(g, buf, sem):
        return pltpu.async_copy(tt_hbm.at[:, pl.ds(g * GW, GW)], buf, sem)

    def transpose_group(src, dst):
        def tdg(d0, _):
            cvec = lax.rem(d0 + iota, D)
            svec = iota * D + cvec

            @plsc.parallel_loop(0, GW // L, unroll=16)
            def _(q):
                val = plsc.load_gather(src, [cvec, iota + q * L])
                plsc.store_scatter(dst, [svec + q * L * D], val)
            return ()
        lax.fori_loop(0, D, tdg, ())

    @pl.when(n > 0)
    def _():
        read(g0, t4a, rsa).wait()

    def step(v, _):
        for lg, tbuf, rbuf, rsem, wsem, rsem_n, tbuf_n in (
                (2 * v, t4a, r4a, rsa, wsa, rsb, t4b),
                (2 * v + 1, t4b, r4b, rsb, wsb, rsa, t4a)):
            g = g0 + lg

            @pl.when(g < gend)
            def _(lg=lg, tbuf=tbuf, rbuf=rbuf, rsem=rsem, wsem=wsem,
                  rsem_n=rsem_n, tbuf_n=tbuf_n, g=g):
                @pl.when(lg > 0)
                def _():
                    pltpu.make_async_copy(
                        tt_hbm.at[:, pl.ds(0, GW)], tbuf, rsem).wait()

                @pl.when(g + 1 < gend)
                def _():
                    read(g + 1, tbuf_n, rsem_n)

                transpose_group(tbuf, rbuf)

                @pl.when(lg >= 2)
                def _():
                    pltpu.make_async_copy(
                        rbuf, tlin4_hbm.at[pl.ds(0, GW * D)], wsem).wait()

                pltpu.async_copy(rbuf,
                                 tlin4_hbm.at[pl.ds(g * GW * D, GW * D)],
                                 wsem)
        return ()
    lax.fori_loop(0, VHALF, step, ())

    @pl.when(n >= 1)
    def _():
        pltpu.make_async_copy(r4a, tlin4_hbm.at[pl.ds(0, GW * D)], wsa).wait()

    @pl.when(n >= 2)
    def _():
        pltpu.make_async_copy(r4b, tlin4_hbm.at[pl.ds(0, GW * D)], wsb).wait()

    @pl.when(w == NW - 1)
    def _():
        pltpu.sync_copy(tail_hbm, tailv)

        def tl(j, _):
            for h in range(2):
                r4a[pl.ds(j * D + L * h, L)] = tailv[j, pl.ds(L * h, L)]
            return ()
        lax.fori_loop(0, W_LAST, tl, (), unroll=4)
        pltpu.sync_copy(r4a.at[pl.ds(0, W_LAST * D)],
                        tlin4_hbm.at[pl.ds(NT_FULL * TILE * D, W_LAST * D)])


def _lookup_body(xtf_hbm, offr_hbm, tlin_hbm, bias_hbm, out_hbm,
                 xv, idx_v, rows_v, obuf, offs_v, bias_v, sem):
    c = lax.axis_index("c")
    s = lax.axis_index("s")
    iota = lax.iota(jnp.int32, L)

    pltpu.sync_copy(offr_hbm, offs_v)
    pltpu.sync_copy(bias_hbm, bias_v.at[pl.ds(0, F * D)])

    b0 = s * BPT
    zeros = jnp.zeros((L,), jnp.int32)

    def feat_body(fi, _):
        f = c * FPC + fi
        off_splat = offs_v[pl.ds(f * L, L)]

        pltpu.sync_copy(xtf_hbm.at[pl.ds(f * B + b0, BPT)], xv)

        @plsc.parallel_loop(0, BPT // L, unroll=8)
        def _(i):
            idx_v[i // 8, pl.ds(lax.rem(i, 8) * L, L)] = \
                xv[pl.ds(i * L, L)] + off_splat

        cps = [pltpu.async_copy(tlin_hbm.at[idx_v.at[j]],
                                rows_v.at[pl.ds(j * 128, 128)], sem)
               for j in range(NGF)]
        for cp in cps:
            cp.wait()

        def td(d0, _):
            cvec = lax.rem(d0 + iota, D)
            bvec = plsc.load_gather(bias_v, [cvec + f * D])
            dvec = cvec // 8 * 8192 + lax.rem(cvec, 8) * 128 + iota

            @plsc.parallel_loop(0, BPT // L, unroll=8)
            def _(q):
                val = plsc.load_gather(rows_v, [iota + q * L, cvec])
                base = q * L // 128 * 1024 + lax.rem(q * L, 128)
                plsc.store_scatter(obuf, [dvec + base], val + bvec)
            return ()
        lax.fori_loop(0, D, td, ())

        for a in range(D // 8):
            pltpu.sync_copy(
                obuf.at[pl.ds(a * 8192, 8192)],
                out_hbm.at[pl.ds((f * 4 + a) * 131072 + s * 8192, 8192)])
        return ()
    lax.fori_loop(0, FPC, feat_body, ())


def _mk_ka():
    mesh = plsc.VectorSubcoreMesh(core_axis_name="c", subcore_axis_name="s")
    return pl.kernel(
        _transpose_body,
        out_type=jax.ShapeDtypeStruct((V * D,), jnp.float32),
        mesh=mesh,
        compiler_params=pltpu.CompilerParams(use_tc_tiling_on_sc=True,
                                             needs_layout_passes=False,
                                             disable_bounds_checks=True),
        scratch_types=[
            pltpu.VMEM((D, GW), jnp.float32),
            pltpu.VMEM((D, GW), jnp.float32),
            pltpu.VMEM((GW * D,), jnp.float32),
            pltpu.VMEM((GW * D,), jnp.float32),
            pltpu.VMEM((W_LAST, TILE), jnp.float32),
            pltpu.SemaphoreType.DMA,
            pltpu.SemaphoreType.DMA,
            pltpu.SemaphoreType.DMA,
            pltpu.SemaphoreType.DMA,
        ],
    )


def _mk_kb():
    mesh = plsc.VectorSubcoreMesh(core_axis_name="c", subcore_axis_name="s")
    return pl.kernel(
        _lookup_body,
        out_type=jax.ShapeDtypeStruct((F * D * B,), jnp.float32),
        mesh=mesh,
        compiler_params=pltpu.CompilerParams(use_tc_tiling_on_sc=False,
                                             needs_layout_passes=False,
                                             disable_bounds_checks=True),
        scratch_types=[
            pltpu.VMEM((BPT,), jnp.int32),
            pltpu.VMEM((NGF, 128), jnp.int32),
            pltpu.VMEM((BPT, D), jnp.float32),
            pltpu.VMEM((D // 8 * BPT // 128 * 8 * 128,), jnp.float32),
            pltpu.VMEM((F * L,), jnp.int32),
            pltpu.VMEM((896,), jnp.float32),
            pltpu.SemaphoreType.DMA,
        ],
    )


@functools.partial(jax.jit, static_argnames=())
def kernel(x, table, bias, offsets):
    tail = jnp.pad(table[V - W_LAST:, :], ((0, 0), (0, TILE - D)))
    tlin1 = _mk_ka()(table.T, tail)
    tlin = tlin1.reshape(V, D)

    offs_rep = jnp.repeat(offsets, L)
    out1 = _mk_kb()(x.T.reshape(-1), offs_rep, tlin, bias.reshape(-1))
    out5 = out1.reshape(F, D // 8, B // 128, 8, 128)
    return out5.transpose(2, 4, 0, 1, 3).reshape(B, F, D)


# speedup vs baseline: 1.0540x; 1.0540x over previous
import functools

import jax
import jax.numpy as jnp
from jax import lax
from jax.experimental import pallas as pl
from jax.experimental.pallas import tpu as pltpu
from jax.experimental.pallas import tpu_sc as plsc

F = 26
D = 32
B = 16384
V = 2600000
NC, NS, L = 2, 16, 16
NW = NC * NS
FPC = F // NC
BPT = B // NS
NGF = BPT // 128

TILE = 128
NT_FULL = V // TILE
W_LAST = V - NT_FULL * TILE
VP = V * D // TILE
GW = 4 * TILE
NGRP = NT_FULL // 4
GPW = -(-NGRP // NW)
VHALF = -(-GPW // 2)


def _transpose_body(tt_hbm, tail_hbm, tlin4_hbm,
                    t4a, t4b, r4a, r4b, tailv, rsa, rsb, wsa, wsb):
    c = lax.axis_index("c")
    s = lax.axis_index("s")
    w = s * NC + c
    iota = lax.iota(jnp.int32, L)

    g0 = w * GPW
    gend = jnp.minimum(g0 + GPW, NGRP)
    n = gend - g0

    def read(g, buf, sem):
        return pltpu.async_copy(tt_hbm.at[:, pl.ds(g * GW, GW)], buf, sem)

    def transpose_group(src, dst):
        def tdg(d0, _):
            cvec = lax.rem(d0 + iota, D)
            svec = iota * D + cvec

            @plsc.parallel_loop(0, GW // L, unroll=8)
            def _(q):
                val = plsc.load_gather(src, [cvec, iota + q * L])
                plsc.store_scatter(dst, [svec + q * L * D], val)
            return ()
        lax.fori_loop(0, D, tdg, ())

    @pl.when(n > 0)
    def _():
        read(g0, t4a, rsa).wait()

    def step(v, _):
        for lg, tbuf, rbuf, rsem, wsem, rsem_n, tbuf_n in (
                (2 * v, t4a, r4a, rsa, wsa, rsb, t4b),
                (2 * v + 1, t4b, r4b, rsb, wsb, rsa, t4a)):
            g = g0 + lg

            @pl.when(g < gend)
            def _(lg=lg, tbuf=tbuf, rbuf=rbuf, rsem=rsem, wsem=wsem,
                  rsem_n=rsem_n, tbuf_n=tbuf_n, g=g):
                @pl.when(lg > 0)
                def _():
                    pltpu.make_async_copy(
                        tt_hbm.at[:, pl.ds(0, GW)], tbuf, rsem).wait()

                @pl.when(g + 1 < gend)
                def _():
                    read(g + 1, tbuf_n, rsem_n)

                transpose_group(tbuf, rbuf)

                @pl.when(lg >= 2)
                def _():
                    pltpu.make_async_copy(
                        rbuf, tlin4_hbm.at[pl.ds(0, GW * D)], wsem).wait()

                pltpu.async_copy(rbuf,
                                 tlin4_hbm.at[pl.ds(g * GW * D, GW * D)],
                                 wsem)
        return ()
    lax.fori_loop(0, VHALF, step, ())

    @pl.when(n >= 1)
    def _():
        pltpu.make_async_copy(r4a, tlin4_hbm.at[pl.ds(0, GW * D)], wsa).wait()

    @pl.when(n >= 2)
    def _():
        pltpu.make_async_copy(r4b, tlin4_hbm.at[pl.ds(0, GW * D)], wsb).wait()

    @pl.when(w == NW - 1)
    def _():
        pltpu.sync_copy(tail_hbm, tailv)

        def tl(j, _):
            for h in range(2):
                r4a[pl.ds(j * D + L * h, L)] = tailv[j, pl.ds(L * h, L)]
            return ()
        lax.fori_loop(0, W_LAST, tl, (), unroll=4)
        pltpu.sync_copy(r4a.at[pl.ds(0, W_LAST * D)],
                        tlin4_hbm.at[pl.ds(NT_FULL * TILE * D, W_LAST * D)])


def _lookup_body(xtf_hbm, offr_hbm, tlin_hbm, bias_hbm, out_hbm,
                 xv, idx_v, rows_v, obuf, offs_v, bias_v, sem):
    c = lax.axis_index("c")
    s = lax.axis_index("s")
    iota = lax.iota(jnp.int32, L)

    pltpu.sync_copy(offr_hbm, offs_v)
    pltpu.sync_copy(bias_hbm, bias_v.at[pl.ds(0, F * D)])

    b0 = s * BPT
    zeros = jnp.zeros((L,), jnp.int32)

    def feat_body(fi, _):
        f = c * FPC + fi
        off_splat = offs_v[pl.ds(f * L, L)]

        pltpu.sync_copy(xtf_hbm.at[pl.ds(f * B + b0, BPT)], xv)

        @plsc.parallel_loop(0, BPT // L, unroll=8)
        def _(i):
            idx_v[i // 8, pl.ds(lax.rem(i, 8) * L, L)] = \
                xv[pl.ds(i * L, L)] + off_splat

        cps = [pltpu.async_copy(tlin_hbm.at[idx_v.at[j]],
                                rows_v.at[pl.ds(j * 128, 128)], sem)
               for j in range(NGF)]
        for cp in cps:
            cp.wait()

        def td(d0, _):
            cvec = lax.rem(d0 + iota, D)
            bvec = plsc.load_gather(bias_v, [cvec + f * D])
            dvec = cvec // 8 * 8192 + lax.rem(cvec, 8) * 128 + iota

            @plsc.parallel_loop(0, BPT // L, unroll=8)
            def _(q):
                val = plsc.load_gather(rows_v, [iota + q * L, cvec])
                base = q * L // 128 * 1024 + lax.rem(q * L, 128)
                plsc.store_scatter(obuf, [dvec + base], val + bvec)
            return ()
        lax.fori_loop(0, D, td, ())

        for a in range(D // 8):
            pltpu.sync_copy(
                obuf.at[pl.ds(a * 8192, 8192)],
                out_hbm.at[pl.ds((f * 4 + a) * 131072 + s * 8192, 8192)])
        return ()
    lax.fori_loop(0, FPC, feat_body, ())


def _mk_ka():
    mesh = plsc.VectorSubcoreMesh(core_axis_name="c", subcore_axis_name="s")
    return pl.kernel(
        _transpose_body,
        out_type=jax.ShapeDtypeStruct((V * D,), jnp.float32),
        mesh=mesh,
        compiler_params=pltpu.CompilerParams(use_tc_tiling_on_sc=True,
                                             needs_layout_passes=False,
                                             disable_bounds_checks=True),
        scratch_types=[
            pltpu.VMEM((D, GW), jnp.float32),
            pltpu.VMEM((D, GW), jnp.float32),
            pltpu.VMEM((GW * D,), jnp.float32),
            pltpu.VMEM((GW * D,), jnp.float32),
            pltpu.VMEM((W_LAST, TILE), jnp.float32),
            pltpu.SemaphoreType.DMA,
            pltpu.SemaphoreType.DMA,
            pltpu.SemaphoreType.DMA,
            pltpu.SemaphoreType.DMA,
        ],
    )


def _mk_kb():
    mesh = plsc.VectorSubcoreMesh(core_axis_name="c", subcore_axis_name="s")
    return pl.kernel(
        _lookup_body,
        out_type=jax.ShapeDtypeStruct((F * D * B,), jnp.float32),
        mesh=mesh,
        compiler_params=pltpu.CompilerParams(use_tc_tiling_on_sc=False,
                                             needs_layout_passes=False,
                                             disable_bounds_checks=True),
        scratch_types=[
            pltpu.VMEM((BPT,), jnp.int32),
            pltpu.VMEM((NGF, 128), jnp.int32),
            pltpu.VMEM((BPT, D), jnp.float32),
            pltpu.VMEM((D // 8 * BPT // 128 * 8 * 128,), jnp.float32),
            pltpu.VMEM((F * L,), jnp.int32),
            pltpu.VMEM((896,), jnp.float32),
            pltpu.SemaphoreType.DMA,
        ],
    )


@functools.partial(jax.jit, static_argnames=())
def kernel(x, table, bias, offsets):
    tail = jnp.pad(table[V - W_LAST:, :], ((0, 0), (0, TILE - D)))
    tlin1 = _mk_ka()(table.T, tail)
    tlin = tlin1.reshape(V, D)

    offs_rep = jnp.repeat(offsets, L)
    out1 = _mk_kb()(x.T.reshape(-1), offs_rep, tlin, bias.reshape(-1))
    out5 = out1.reshape(F, D // 8, B // 128, 8, 128)
    return out5.transpose(2, 4, 0, 1, 3).reshape(B, F, D)
